# baseline (speedup 1.0000x reference)
import functools

import numpy as np
import jax
import jax.numpy as jnp
from jax import lax
from jax.experimental import pallas as pl
from jax.experimental.pallas import tpu as pltpu

SELU_ALPHA = np.float32(1.6732632423543772848170429916717)
SELU_SCALE = np.float32(1.0507009873554804934193349852946)
BN_EPS = 1e-5
KEEP_P = np.float32(0.8)
INV_KEEP = np.float32(1.25)
TF_PARITY = 0x1BD11BDA
FLOAT_ONE = 0x3F800000


def _round_up(a, b):
    return -(-a // b) * b


def _conv_windows(xblk, c_in, l):
    tile_n = xblk.shape[0]
    zc = jnp.zeros((tile_n, 1), jnp.float32)
    wins = []
    for ci in range(c_in):
        xi = xblk[:, ci * l:(ci + 1) * l]
        xm = jnp.concatenate([zc, xi[:, :l - 1]], axis=1)
        xp = jnp.concatenate([xi[:, 1:], zc], axis=1)
        wins.append((xm, xi, xp))
    return wins


def _conv_channel(wins, w_ref, co, c_in):
    acc = None
    for ci in range(c_in):
        for k in range(3):
            term = w_ref[(co * c_in + ci) * 3 + k] * wins[ci][k]
            acc = term if acc is None else acc + term
    return acc


def _threefry_bits(cnt, ks1, ks2):
    rot_a = (13, 15, 26, 6)
    rot_b = (17, 29, 16, 24)

    def rotl(v, r):
        return lax.shift_left(v, np.uint32(r)) | lax.shift_right_logical(
            v, np.uint32(32 - r))

    def rounds(x0, x1, rots):
        for r in rots:
            x0 = x0 + x1
            x1 = rotl(x1, r)
            x1 = x0 ^ x1
        return x0, x1

    x1 = cnt + ks1
    x0 = x1
    x1 = rotl(x1, 13) ^ x0
    x0, x1 = rounds(x0, x1, rot_a[1:])
    x0, x1 = x0 + ks1, x1 + (ks2 + np.uint32(1))
    x0, x1 = rounds(x0, x1, rot_b)
    x0, x1 = x0 + ks2, x1 + np.uint32(2)
    x0, x1 = rounds(x0, x1, rot_a)
    x0, x1 = x0, x1 + (ks1 + np.uint32(3))
    x0, x1 = rounds(x0, x1, rot_b)
    x0, x1 = x0 + ks1, x1 + (ks2 + np.uint32(4))
    x0, x1 = rounds(x0, x1, rot_a)
    x0, x1 = x0 + ks2, x1 + np.uint32(5)
    return x0 ^ x1


def _stats_body(x_ref, w_ref, sum_ref, sq_ref, *, c_in, c_out, l):
    @pl.when(pl.program_id(1) == 0)
    def _init():
        sum_ref[...] = jnp.zeros_like(sum_ref)
        sq_ref[...] = jnp.zeros_like(sq_ref)

    wins = _conv_windows(x_ref[0], c_in, l)
    for co in range(c_out):
        y = _conv_channel(wins, w_ref, co, c_in)
        sum_ref[0, co, :] += jnp.sum(y, axis=0)
        sq_ref[0, co, :] += jnp.sum(y * y, axis=0)


def _apply_body(x_ref, w_ref, shift_ref, seed_ref, o_ref, *,
                c_in, c_out, l, tile_n, blocks_per_slice, row_shift):
    blk = pl.program_id(0) * blocks_per_slice + pl.program_id(1)
    row0 = blk * tile_n

    seed = seed_ref[0]
    ks1 = seed
    ks2 = seed ^ np.uint32(TF_PARITY)

    rows = lax.broadcasted_iota(jnp.int32, (tile_n, l), 0) + row0
    cols = lax.broadcasted_iota(jnp.int32, (tile_n, l), 1)
    if row_shift is not None:
        idx0 = lax.shift_left(rows, row_shift) + cols
    else:
        idx0 = rows * (c_out * l) + cols

    wins = _conv_windows(x_ref[0], c_in, l)
    for co in range(c_out):
        y = _conv_channel(wins, w_ref, co, c_in) + shift_ref[co]
        bits = _threefry_bits((idx0 + co * l).astype(jnp.uint32), ks1, ks2)
        fbits = lax.shift_right_logical(bits, np.uint32(9)) | np.uint32(FLOAT_ONE)
        u = lax.bitcast_convert_type(fbits, jnp.float32) - np.float32(1.0)
        y = jnp.where(u < KEEP_P, y * INV_KEEP, np.float32(0.0))
        o_ref[0, :, co * l:(co + 1) * l] = SELU_SCALE * jnp.where(
            y > np.float32(0.0), y, SELU_ALPHA * (jnp.exp(y) - np.float32(1.0)))


@functools.partial(jax.jit, static_argnames=("stats_tile", "apply_tile"))
def _conv_bn_drop_selu(x, weight, gamma, beta, seed, *,
                       stats_tile, apply_tile):
    N, C_in, L = x.shape
    C_out = weight.shape[0]
    L_out = L

    x2 = x.reshape(N, C_in * L).astype(jnp.float32)
    w_flat = weight.reshape(C_out * C_in * 3).astype(jnp.float32)
    smem = pl.BlockSpec(memory_space=pltpu.MemorySpace.SMEM)
    conv_flops = 2 * N * L_out * C_out * C_in * 3

    s_blocks = N // stats_tile
    s_slices = 2 if s_blocks % 2 == 0 else 1
    s_per = s_blocks // s_slices
    acc_spec = pl.BlockSpec((1, C_out, L_out), lambda s, i: (s, 0, 0))
    psum, psq = pl.pallas_call(
        functools.partial(_stats_body, c_in=C_in, c_out=C_out, l=L),
        grid=(s_slices, s_per),
        in_specs=[pl.BlockSpec((1, stats_tile, C_in * L),
                               lambda s, i: (0, s * s_per + i, 0)),
                  smem],
        out_specs=[acc_spec, acc_spec],
        out_shape=[jax.ShapeDtypeStruct((s_slices, C_out, L_out),
                                        jnp.float32)] * 2,
        compiler_params=pltpu.CompilerParams(
            dimension_semantics=("parallel", "arbitrary")),
        cost_estimate=pl.CostEstimate(
            flops=conv_flops + 3 * N * C_out * L_out,
            transcendentals=0,
            bytes_accessed=4 * (N * C_in * L + w_flat.size
                                + 2 * s_slices * C_out * L_out)),
    )(x2.reshape(1, N, C_in * L), w_flat)

    count = N * L_out
    mean = psum.sum(axis=(0, 2)) / count
    var = jnp.maximum(psq.sum(axis=(0, 2)) / count - mean * mean, 0.0)
    inv_std = lax.rsqrt(var + BN_EPS)
    scale = gamma.astype(jnp.float32) * inv_std
    shift = beta.astype(jnp.float32) - mean * scale
    w_scaled = (w_flat.reshape(C_out, C_in * 3) * scale[:, None]).reshape(-1)

    a_blocks = N // apply_tile
    a_slices = 2 if a_blocks % 2 == 0 else 1
    a_per = a_blocks // a_slices
    row_width = C_out * L_out
    row_shift = int(row_width).bit_length() - 1
    if (1 << row_shift) != row_width:
        row_shift = None
    seed_u = jnp.asarray(seed, jnp.uint32).reshape(1)

    out = pl.pallas_call(
        functools.partial(_apply_body, c_in=C_in, c_out=C_out, l=L,
                          tile_n=apply_tile, blocks_per_slice=a_per,
                          row_shift=row_shift),
        grid=(a_slices, a_per),
        in_specs=[pl.BlockSpec((1, apply_tile, C_in * L),
                               lambda s, i: (0, s * a_per + i, 0)),
                  smem, smem, smem],
        out_specs=pl.BlockSpec((1, apply_tile, row_width),
                               lambda s, i: (0, s * a_per + i, 0)),
        out_shape=jax.ShapeDtypeStruct((1, N, row_width), jnp.float32),
        compiler_params=pltpu.CompilerParams(
            dimension_semantics=("parallel", "arbitrary")),
        cost_estimate=pl.CostEstimate(
            flops=conv_flops + 130 * N * row_width,
            transcendentals=N * row_width,
            bytes_accessed=4 * (N * C_in * L + N * row_width
                                + w_flat.size + 2 * C_out)),
    )(x2.reshape(1, N, C_in * L), w_scaled, shift, seed_u)

    return out.reshape(N, C_out, L_out)


def _pick_tile(n, want):
    t = min(want, n)
    while n % t:
        t -= 8
    return max(t, 8)


def kernel(x, weight, bias, gamma, beta, seed):
    del bias
    N = x.shape[0]
    return _conv_bn_drop_selu(
        x, weight, gamma, beta, seed,
        stats_tile=_pick_tile(N, 16), apply_tile=_pick_tile(N, 16))


# speedup vs baseline: 1.0115x; 1.0115x over previous
import functools

import numpy as np
import jax
import jax.numpy as jnp
from jax import lax
from jax.experimental import pallas as pl
from jax.experimental.pallas import tpu as pltpu

SELU_ALPHA = np.float32(1.6732632423543772848170429916717)
SELU_SCALE = np.float32(1.0507009873554804934193349852946)
BN_EPS = 1e-5
KEEP_P = np.float32(0.8)
INV_KEEP = np.float32(1.25)
TF_PARITY = 0x1BD11BDA
FLOAT_ONE = 0x3F800000


def _round_up(a, b):
    return -(-a // b) * b


def _conv_windows(xblk, c_in, l):
    tile_n = xblk.shape[0]
    zc = jnp.zeros((tile_n, 1), jnp.float32)
    wins = []
    for ci in range(c_in):
        xi = xblk[:, ci * l:(ci + 1) * l]
        xm = jnp.concatenate([zc, xi[:, :l - 1]], axis=1)
        xp = jnp.concatenate([xi[:, 1:], zc], axis=1)
        wins.append((xm, xi, xp))
    return wins


def _conv_channel(wins, w_ref, co, c_in):
    acc = None
    for ci in range(c_in):
        for k in range(3):
            term = w_ref[(co * c_in + ci) * 3 + k] * wins[ci][k]
            acc = term if acc is None else acc + term
    return acc


def _threefry_bits(cnt, ks1, ks2):
    rot_a = (13, 15, 26, 6)
    rot_b = (17, 29, 16, 24)

    def rotl(v, r):
        return lax.shift_left(v, np.uint32(r)) | lax.shift_right_logical(
            v, np.uint32(32 - r))

    def rounds(x0, x1, rots):
        for r in rots:
            x0 = x0 + x1
            x1 = rotl(x1, r)
            x1 = x0 ^ x1
        return x0, x1

    x1 = cnt + ks1
    x0 = x1
    x1 = rotl(x1, 13) ^ x0
    x0, x1 = rounds(x0, x1, rot_a[1:])
    x0, x1 = x0 + ks1, x1 + (ks2 + np.uint32(1))
    x0, x1 = rounds(x0, x1, rot_b)
    x0, x1 = x0 + ks2, x1 + np.uint32(2)
    x0, x1 = rounds(x0, x1, rot_a)
    x0, x1 = x0, x1 + (ks1 + np.uint32(3))
    x0, x1 = rounds(x0, x1, rot_b)
    x0, x1 = x0 + ks1, x1 + (ks2 + np.uint32(4))
    x0, x1 = rounds(x0, x1, rot_a)
    x0, x1 = x0 + ks2, x1 + np.uint32(5)
    return x0 ^ x1


def _stats_body(x_ref, w_ref, sum_ref, sq_ref, *, c_in, c_out, l):
    @pl.when(pl.program_id(1) == 0)
    def _init():
        sum_ref[...] = jnp.zeros_like(sum_ref)
        sq_ref[...] = jnp.zeros_like(sq_ref)

    wins = _conv_windows(x_ref[0], c_in, l)
    for co in range(c_out):
        y = _conv_channel(wins, w_ref, co, c_in)
        sum_ref[0, co, :] += jnp.sum(y, axis=0)
        sq_ref[0, co, :] += jnp.sum(y * y, axis=0)


def _apply_body(x_ref, w_ref, shift_ref, seed_ref, o_ref, *,
                c_in, c_out, l, tile_n, blocks_per_slice, row_shift):
    blk = pl.program_id(0) * blocks_per_slice + pl.program_id(1)
    row0 = blk * tile_n

    seed = seed_ref[0]
    ks1 = seed
    ks2 = seed ^ np.uint32(TF_PARITY)

    rows = lax.broadcasted_iota(jnp.int32, (tile_n, l), 0) + row0
    cols = lax.broadcasted_iota(jnp.int32, (tile_n, l), 1)
    if row_shift is not None:
        idx0 = lax.shift_left(rows, row_shift) + cols
    else:
        idx0 = rows * (c_out * l) + cols

    wins = _conv_windows(x_ref[0], c_in, l)
    for co in range(c_out):
        y = _conv_channel(wins, w_ref, co, c_in) + shift_ref[co]
        bits = _threefry_bits((idx0 + co * l).astype(jnp.uint32), ks1, ks2)
        fbits = lax.shift_right_logical(bits, np.uint32(9)) | np.uint32(FLOAT_ONE)
        u = lax.bitcast_convert_type(fbits, jnp.float32) - np.float32(1.0)
        y = jnp.where(u < KEEP_P, y * INV_KEEP, np.float32(0.0))
        o_ref[0, :, co * l:(co + 1) * l] = SELU_SCALE * jnp.where(
            y > np.float32(0.0), y, SELU_ALPHA * (jnp.exp(y) - np.float32(1.0)))


@functools.partial(jax.jit, static_argnames=("stats_tile", "apply_tile"))
def _conv_bn_drop_selu(x, weight, gamma, beta, seed, *,
                       stats_tile, apply_tile):
    N, C_in, L = x.shape
    C_out = weight.shape[0]
    L_out = L

    x2 = x.reshape(N, C_in * L).astype(jnp.float32)
    w_flat = weight.reshape(C_out * C_in * 3).astype(jnp.float32)
    smem = pl.BlockSpec(memory_space=pltpu.MemorySpace.SMEM)
    conv_flops = 2 * N * L_out * C_out * C_in * 3

    s_blocks = N // stats_tile
    s_slices = 2 if s_blocks % 2 == 0 else 1
    s_per = s_blocks // s_slices
    acc_spec = pl.BlockSpec((1, C_out, L_out), lambda s, i: (s, 0, 0))
    psum, psq = pl.pallas_call(
        functools.partial(_stats_body, c_in=C_in, c_out=C_out, l=L),
        grid=(s_slices, s_per),
        in_specs=[pl.BlockSpec((1, stats_tile, C_in * L),
                               lambda s, i: (0, s * s_per + i, 0)),
                  smem],
        out_specs=[acc_spec, acc_spec],
        out_shape=[jax.ShapeDtypeStruct((s_slices, C_out, L_out),
                                        jnp.float32)] * 2,
        compiler_params=pltpu.CompilerParams(
            dimension_semantics=("parallel", "arbitrary")),
        cost_estimate=pl.CostEstimate(
            flops=conv_flops + 3 * N * C_out * L_out,
            transcendentals=0,
            bytes_accessed=4 * (N * C_in * L + w_flat.size
                                + 2 * s_slices * C_out * L_out)),
    )(x2.reshape(1, N, C_in * L), w_flat)

    count = N * L_out
    mean = psum.sum(axis=(0, 2)) / count
    var = jnp.maximum(psq.sum(axis=(0, 2)) / count - mean * mean, 0.0)
    inv_std = lax.rsqrt(var + BN_EPS)
    scale = gamma.astype(jnp.float32) * inv_std
    shift = beta.astype(jnp.float32) - mean * scale
    w_scaled = (w_flat.reshape(C_out, C_in * 3) * scale[:, None]).reshape(-1)

    a_blocks = N // apply_tile
    a_slices = 2 if a_blocks % 2 == 0 else 1
    a_per = a_blocks // a_slices
    row_width = C_out * L_out
    row_shift = int(row_width).bit_length() - 1
    if (1 << row_shift) != row_width:
        row_shift = None
    seed_u = jnp.asarray(seed, jnp.uint32).reshape(1)

    out = pl.pallas_call(
        functools.partial(_apply_body, c_in=C_in, c_out=C_out, l=L,
                          tile_n=apply_tile, blocks_per_slice=a_per,
                          row_shift=row_shift),
        grid=(a_slices, a_per),
        in_specs=[pl.BlockSpec((1, apply_tile, C_in * L),
                               lambda s, i: (0, s * a_per + i, 0)),
                  smem, smem, smem],
        out_specs=pl.BlockSpec((1, apply_tile, row_width),
                               lambda s, i: (0, s * a_per + i, 0)),
        out_shape=jax.ShapeDtypeStruct((1, N, row_width), jnp.float32),
        compiler_params=pltpu.CompilerParams(
            dimension_semantics=("parallel", "arbitrary")),
        cost_estimate=pl.CostEstimate(
            flops=conv_flops + 130 * N * row_width,
            transcendentals=N * row_width,
            bytes_accessed=4 * (N * C_in * L + N * row_width
                                + w_flat.size + 2 * C_out)),
    )(x2.reshape(1, N, C_in * L), w_scaled, shift, seed_u)

    return out.reshape(N, C_out, L_out)


def _pick_tile(n, want):
    t = min(want, n)
    while n % t:
        t -= 8
    return max(t, 8)


def kernel(x, weight, bias, gamma, beta, seed):
    del bias
    N = x.shape[0]
    return _conv_bn_drop_selu(
        x, weight, gamma, beta, seed,
        stats_tile=_pick_tile(N, 16), apply_tile=_pick_tile(N, 8))


# speedup vs baseline: 1.7727x; 1.7525x over previous
import functools

import numpy as np
import jax
import jax.numpy as jnp
from jax import lax
from jax.experimental import pallas as pl
from jax.experimental.pallas import tpu as pltpu
from jax.sharding import Mesh, PartitionSpec as P

try:
    from jax import shard_map as _shard_map_fn

    def _shard_map(f, mesh, in_specs, out_specs):
        return _shard_map_fn(f, mesh=mesh, in_specs=in_specs,
                             out_specs=out_specs, check_vma=False)
except ImportError:
    from jax.experimental.shard_map import shard_map as _shard_map_legacy

    def _shard_map(f, mesh, in_specs, out_specs):
        return _shard_map_legacy(f, mesh=mesh, in_specs=in_specs,
                                 out_specs=out_specs, check_rep=False)

SELU_ALPHA = np.float32(1.6732632423543772848170429916717)
SELU_SCALE = np.float32(1.0507009873554804934193349852946)
BN_EPS = 1e-5
KEEP_P = np.float32(0.8)
INV_KEEP = np.float32(1.25)
TF_PARITY = 0x1BD11BDA
KEEP_THRESH = np.uint32(6710887 << 9)


def _conv_windows(xblk, c_in, l):
    tile_n = xblk.shape[0]
    zc = jnp.zeros((tile_n, 1), jnp.float32)
    wins = []
    for ci in range(c_in):
        xi = xblk[:, ci * l:(ci + 1) * l]
        xm = jnp.concatenate([zc, xi[:, :l - 1]], axis=1)
        xp = jnp.concatenate([xi[:, 1:], zc], axis=1)
        wins.append((xm, xi, xp))
    return wins


def _conv_channel(wins, w_ref, co, c_in):
    acc = None
    for ci in range(c_in):
        for k in range(3):
            term = w_ref[(co * c_in + ci) * 3 + k] * wins[ci][k]
            acc = term if acc is None else acc + term
    return acc


def _threefry_bits(cnt, ks1, ks2):
    rot_a = (13, 15, 26, 6)
    rot_b = (17, 29, 16, 24)

    def rotl(v, r):
        return lax.shift_left(v, np.uint32(r)) | lax.shift_right_logical(
            v, np.uint32(32 - r))

    def rounds(x0, x1, rots):
        for r in rots:
            x0 = x0 + x1
            x1 = rotl(x1, r)
            x1 = x0 ^ x1
        return x0, x1

    x1 = cnt + ks1
    x0 = x1
    x1 = rotl(x1, 13) ^ x0
    x0, x1 = rounds(x0, x1, rot_a[1:])
    x0, x1 = x0 + ks1, x1 + (ks2 + np.uint32(1))
    x0, x1 = rounds(x0, x1, rot_b)
    x0, x1 = x0 + ks2, x1 + np.uint32(2)
    x0, x1 = rounds(x0, x1, rot_a)
    x0, x1 = x0, x1 + (ks1 + np.uint32(3))
    x0, x1 = rounds(x0, x1, rot_b)
    x0, x1 = x0 + ks1, x1 + (ks2 + np.uint32(4))
    x0, x1 = rounds(x0, x1, rot_a)
    x0, x1 = x0 + ks2, x1 + np.uint32(5)
    return x0 ^ x1


def _stats_body(x_ref, w_ref, sum_ref, sq_ref, *, c_in, c_out, l):
    @pl.when(pl.program_id(0) == 0)
    def _init():
        sum_ref[...] = jnp.zeros_like(sum_ref)
        sq_ref[...] = jnp.zeros_like(sq_ref)

    wins = _conv_windows(x_ref[...], c_in, l)
    for co in range(c_out):
        y = _conv_channel(wins, w_ref, co, c_in)
        sum_ref[co, :] += jnp.sum(y, axis=0)
        sq_ref[co, :] += jnp.sum(y * y, axis=0)


def _apply_body(x_ref, w_ref, shift_ref, seed_ref, base_ref, o_ref, *,
                c_in, c_out, l, tile_n, row_shift):
    row0 = base_ref[0] + pl.program_id(0) * tile_n

    seed = seed_ref[0]
    ks1 = seed
    ks2 = seed ^ np.uint32(TF_PARITY)

    rows = lax.broadcasted_iota(jnp.int32, (tile_n, l), 0) + row0
    cols = lax.broadcasted_iota(jnp.int32, (tile_n, l), 1)
    if row_shift is not None:
        idx0 = lax.shift_left(rows, row_shift) + cols
    else:
        idx0 = rows * (c_out * l) + cols

    wins = _conv_windows(x_ref[...], c_in, l)
    for co in range(c_out):
        y = _conv_channel(wins, w_ref, co, c_in) + shift_ref[co]
        bits = _threefry_bits((idx0 + co * l).astype(jnp.uint32), ks1, ks2)
        y = jnp.where(bits < KEEP_THRESH, y, np.float32(0.0))
        o_ref[:, co * l:(co + 1) * l] = SELU_SCALE * jnp.where(
            y > np.float32(0.0), y, SELU_ALPHA * (jnp.exp(y) - np.float32(1.0)))


def _run_shard(x2, w_flat, gamma, beta, seed_u, base_rows, *,
               n_total, stats_tile, apply_tile, axis_name):
    n_loc = x2.shape[0]
    c_in_l = x2.shape[1]
    c_out = gamma.shape[0]
    c_in = w_flat.shape[0] // (c_out * 3)
    l = c_in_l // c_in
    row_width = c_out * l

    smem = pl.BlockSpec(memory_space=pltpu.MemorySpace.SMEM)
    conv_flops = 2 * n_loc * l * c_out * c_in * 3

    acc_spec = pl.BlockSpec((c_out, l), lambda i: (0, 0))
    psum, psq = pl.pallas_call(
        functools.partial(_stats_body, c_in=c_in, c_out=c_out, l=l),
        grid=(n_loc // stats_tile,),
        in_specs=[pl.BlockSpec((stats_tile, c_in_l), lambda i: (i, 0)),
                  smem],
        out_specs=[acc_spec, acc_spec],
        out_shape=[jax.ShapeDtypeStruct((c_out, l), jnp.float32)] * 2,
        compiler_params=pltpu.CompilerParams(
            dimension_semantics=("arbitrary",)),
        cost_estimate=pl.CostEstimate(
            flops=conv_flops + 3 * n_loc * row_width,
            transcendentals=0,
            bytes_accessed=4 * (n_loc * c_in_l + w_flat.size
                                + 2 * row_width)),
    )(x2, w_flat)

    ch_sum = psum.sum(axis=1)
    ch_sumsq = psq.sum(axis=1)
    if axis_name is not None:
        ch_sum = lax.psum(ch_sum, axis_name)
        ch_sumsq = lax.psum(ch_sumsq, axis_name)

    count = n_total * l
    mean = ch_sum / count
    var = jnp.maximum(ch_sumsq / count - mean * mean, 0.0)
    inv_std = lax.rsqrt(var + BN_EPS)
    scale = gamma.astype(jnp.float32) * inv_std
    shift = beta.astype(jnp.float32) - mean * scale
    w_scaled = (w_flat.reshape(c_out, c_in * 3)
                * (scale * INV_KEEP)[:, None]).reshape(-1)
    shift = shift * INV_KEEP

    row_shift = int(row_width).bit_length() - 1
    if (1 << row_shift) != row_width:
        row_shift = None

    out = pl.pallas_call(
        functools.partial(_apply_body, c_in=c_in, c_out=c_out, l=l,
                          tile_n=apply_tile, row_shift=row_shift),
        grid=(n_loc // apply_tile,),
        in_specs=[pl.BlockSpec((apply_tile, c_in_l), lambda i: (i, 0)),
                  smem, smem, smem, smem],
        out_specs=pl.BlockSpec((apply_tile, row_width), lambda i: (i, 0)),
        out_shape=jax.ShapeDtypeStruct((n_loc, row_width), jnp.float32),
        compiler_params=pltpu.CompilerParams(
            dimension_semantics=("arbitrary",)),
        cost_estimate=pl.CostEstimate(
            flops=conv_flops + 130 * n_loc * row_width,
            transcendentals=n_loc * row_width,
            bytes_accessed=4 * (n_loc * c_in_l + n_loc * row_width
                                + w_flat.size + 2 * c_out)),
    )(x2, w_scaled, shift, seed_u, base_rows)

    return out


@functools.partial(jax.jit,
                   static_argnames=("stats_tile", "apply_tile", "n_shards"))
def _conv_bn_drop_selu(x, weight, gamma, beta, seed, *,
                       stats_tile, apply_tile, n_shards):
    N, C_in, L = x.shape
    C_out = weight.shape[0]

    x2 = x.reshape(N, C_in * L).astype(jnp.float32)
    w_flat = weight.reshape(C_out * C_in * 3).astype(jnp.float32)
    seed_u = jnp.asarray(seed, jnp.uint32).reshape(1)

    if n_shards > 1:
        mesh = Mesh(np.asarray(jax.devices()[:n_shards]), ("d",))
        n_loc = N // n_shards

        def shard_fn(x2s, wf, g, b, su):
            base = (lax.axis_index("d") * n_loc).astype(jnp.int32).reshape(1)
            return _run_shard(x2s, wf, g, b, su, base,
                              n_total=N, stats_tile=stats_tile,
                              apply_tile=apply_tile, axis_name="d")

        out = _shard_map(
            shard_fn, mesh,
            in_specs=(P("d"), P(), P(), P(), P()),
            out_specs=P("d"))(x2, w_flat, gamma, beta, seed_u)
    else:
        base = jnp.zeros((1,), jnp.int32)
        out = _run_shard(x2, w_flat, gamma, beta, seed_u, base,
                         n_total=N, stats_tile=stats_tile,
                         apply_tile=apply_tile, axis_name=None)

    return out.reshape(N, C_out, L)


def _pick_tile(n, want):
    t = min(want, n)
    while n % t:
        t -= 8
    return max(t, 8)


def kernel(x, weight, bias, gamma, beta, seed):
    del bias
    N = x.shape[0]
    n_shards = 2 if (jax.device_count() >= 2 and N % 2 == 0) else 1
    n_loc = N // n_shards
    return _conv_bn_drop_selu(
        x, weight, gamma, beta, seed,
        stats_tile=_pick_tile(n_loc, 16), apply_tile=_pick_tile(n_loc, 8),
        n_shards=n_shards)
